# baseline (speedup 1.0000x reference)
import functools

import jax
import jax.numpy as jnp
import numpy as np
from jax import lax
from jax.experimental import pallas as pl
from jax.experimental.pallas import tpu as pltpu
from jax.experimental.pallas import tpu_sc as plsc

_NC = 2
_NS = 16


def kernel(inputs, dist_embedding):
    batch, seq = inputs.shape[0], inputs.shape[1]
    emb = dist_embedding.shape[1]
    total = seq * (seq + 1) // 2
    nrows = batch * total
    nw = _NC * _NS
    per_w = nrows // nw
    assert per_w * nw == nrows and per_w % 8 == 0

    chunk = 64
    nfull = per_w // chunk
    tail = per_w - nfull * chunk
    npair = nfull // 2

    idx_np = np.concatenate(
        [np.arange(n, dtype=np.int32) for n in range(seq, 0, -1)])
    idx_all = jnp.asarray(np.tile(idx_np, batch))

    mesh = plsc.VectorSubcoreMesh(core_axis_name="c", subcore_axis_name="s")

    @functools.partial(
        pl.kernel,
        mesh=mesh,
        out_type=jax.ShapeDtypeStruct((nrows, emb), jnp.float32),
        scratch_types=[
            pltpu.VMEM((per_w,), jnp.int32),
            pltpu.VMEM((chunk, emb), jnp.float32),
            pltpu.VMEM((chunk, emb), jnp.float32),
            pltpu.SemaphoreType.DMA,
            pltpu.SemaphoreType.DMA,
            pltpu.SemaphoreType.DMA,
            pltpu.SemaphoreType.DMA,
        ],
        compiler_params=pltpu.CompilerParams(use_tc_tiling_on_sc=False),
    )
    def _gather_kernel(table_hbm, idx_hbm, out_hbm, idx_v,
                       buf0, buf1, g0, g1, s0, s1):
        sid = lax.axis_index("s")
        wid = lax.axis_index("c") * _NS + sid
        base = wid * per_w

        pltpu.sync_copy(idx_hbm.at[pl.ds(base, per_w)], idx_v)

        def gather(c, buf, sem):
            return pltpu.async_copy(
                table_hbm.at[idx_v.at[pl.ds(c * chunk, chunk)]], buf, sem)

        def scatter(c, buf, sem):
            return pltpu.async_copy(
                buf, out_hbm.at[pl.ds(base + c * chunk, chunk)], sem)

        def wait_chunk(buf, sem):
            pltpu.make_async_copy(table_hbm.at[pl.ds(0, chunk)], buf, sem).wait()

        gather(0, buf0, g0)
        gather(1, buf1, g1)

        def body(j, carry):
            c0 = 2 * j
            wait_chunk(buf0, g0)
            scatter(c0, buf0, s0)
            wait_chunk(buf1, g1)
            scatter(c0 + 1, buf1, s1)

            @pl.when(j < npair - 1)
            def _():
                wait_chunk(buf0, s0)
                gather(c0 + 2, buf0, g0)
                wait_chunk(buf1, s1)
                gather(c0 + 3, buf1, g1)

            return carry

        lax.fori_loop(0, npair, body, 0)

        wait_chunk(buf0, s0)
        wait_chunk(buf1, s1)
        toff = nfull * chunk
        pltpu.async_copy(
            table_hbm.at[idx_v.at[pl.ds(toff, tail)]],
            buf0.at[pl.ds(0, tail)], g0).wait()
        pltpu.async_copy(
            buf0.at[pl.ds(0, tail)],
            out_hbm.at[pl.ds(base + toff, tail)], s0).wait()

    out = _gather_kernel(dist_embedding, idx_all)
    return out.reshape(batch, total, emb)


# speedup vs baseline: 1.5008x; 1.5008x over previous
import functools

import jax
import jax.numpy as jnp
from jax import lax
from jax.experimental import pallas as pl
from jax.experimental.pallas import tpu as pltpu
from jax.experimental.pallas import tpu_sc as plsc

_NC = 2
_NS = 16
_WIN = 128


def _balance(items, nbins):
    bins = [[] for _ in range(nbins)]
    loads = [0] * nbins
    for size, payload in sorted(items, key=lambda t: (-t[0], t[1])):
        i = loads.index(min(loads))
        bins[i].append((size, payload))
        loads[i] += size
    return bins


def kernel(inputs, dist_embedding):
    batch, seq = inputs.shape[0], inputs.shape[1]
    emb = dist_embedding.shape[1]
    total = seq * (seq + 1) // 2
    nrows = batch * total
    assert batch == 2 and seq == 2 * _WIN

    starts = [k * seq - (k * (k - 1)) // 2 for k in range(seq)]

    items_w0 = [(min(seq - k, _WIN), starts[k]) for k in range(seq)]
    items_w1 = [(seq - k - _WIN, starts[k] + _WIN) for k in range(_WIN)
                if seq - k > _WIN]

    n_g0, n_g1 = 12, 4
    groups = _balance(items_w0, n_g0) + _balance(items_w1, n_g1)

    mesh = plsc.VectorSubcoreMesh(core_axis_name="c", subcore_axis_name="s")

    @functools.partial(
        pl.kernel,
        mesh=mesh,
        out_type=jax.ShapeDtypeStruct((nrows, emb), jnp.float32),
        scratch_types=[
            pltpu.VMEM((_WIN, emb), jnp.float32),
            pltpu.SemaphoreType.DMA,
        ],
        compiler_params=pltpu.CompilerParams(use_tc_tiling_on_sc=False),
    )
    def _copy_kernel(table_hbm, out_hbm, tbuf, sem):
        wid = lax.axis_index("c") * _NS + lax.axis_index("s")
        g = wid // batch
        b_off = (wid % batch) * total

        win_off = jnp.where(g >= n_g0, _WIN, 0)
        pltpu.sync_copy(table_hbm.at[pl.ds(win_off, _WIN)], tbuf)

        for G, items in enumerate(groups):
            @pl.when(g == G)
            def _(items=items):
                copies = [
                    pltpu.async_copy(
                        tbuf.at[pl.ds(0, L)],
                        out_hbm.at[pl.ds(b_off + dst, L)],
                        sem,
                    )
                    for L, dst in items
                ]
                for c in copies:
                    c.wait()

    out = _copy_kernel(dist_embedding)
    return out.reshape(batch, total, emb)


# speedup vs baseline: 1.5115x; 1.0071x over previous
import functools

import jax
import jax.numpy as jnp
from jax import lax
from jax.experimental import pallas as pl
from jax.experimental.pallas import tpu as pltpu
from jax.experimental.pallas import tpu_sc as plsc

_NC = 2
_NS = 16
_WIN = 128


def _balance(items, nbins):
    bins = [[] for _ in range(nbins)]
    loads = [0] * nbins
    for size, payload in sorted(items, key=lambda t: (-t[0], t[1])):
        i = loads.index(min(loads))
        bins[i].append((size, payload))
        loads[i] += size
    return bins


def kernel(inputs, dist_embedding):
    batch, seq = inputs.shape[0], inputs.shape[1]
    emb = dist_embedding.shape[1]
    total = seq * (seq + 1) // 2
    nrows = batch * total
    assert batch == 2 and seq == 2 * _WIN

    starts = [k * seq - (k * (k - 1)) // 2 for k in range(seq)]

    items_w0 = [(min(seq - k, _WIN), starts[k]) for k in range(seq)]
    items_w1 = [(seq - k - _WIN, starts[k] + _WIN) for k in range(_WIN)
                if seq - k > _WIN]

    n_g0, n_g1 = 12, 4
    groups = _balance(items_w0, n_g0) + _balance(items_w1, n_g1)

    mesh = plsc.VectorSubcoreMesh(core_axis_name="c", subcore_axis_name="s")

    @functools.partial(
        pl.kernel,
        mesh=mesh,
        out_type=jax.ShapeDtypeStruct((nrows * emb,), jnp.float32),
        scratch_types=[
            pltpu.VMEM((_WIN * emb,), jnp.float32),
            pltpu.SemaphoreType.DMA,
        ],
    )
    def _copy_kernel(table_hbm, out_hbm, tbuf, sem):
        wid = lax.axis_index("c") * _NS + lax.axis_index("s")
        g = wid // batch
        b_off = (wid % batch) * (total * emb)

        win_off = jnp.where(g >= n_g0, _WIN * emb, 0)
        pltpu.sync_copy(table_hbm.at[pl.ds(win_off, _WIN * emb)], tbuf)

        for G, items in enumerate(groups):
            @pl.when(g == G)
            def _(items=items):
                copies = [
                    pltpu.async_copy(
                        tbuf.at[pl.ds(0, L * emb)],
                        out_hbm.at[pl.ds(b_off + dst * emb, L * emb)],
                        sem,
                    )
                    for L, dst in items
                ]
                for c in copies:
                    c.wait()

    out = _copy_kernel(dist_embedding.reshape(-1))
    return out.reshape(batch, total, emb)


# speedup vs baseline: 3.2548x; 2.1533x over previous
import functools

import jax
import jax.numpy as jnp
import numpy as np
from jax import lax
from jax.experimental import pallas as pl
from jax.experimental.pallas import tpu as pltpu
from jax.experimental.pallas import tpu_sc as plsc

_NC = 2
_NS = 16
_Q = 64
_BR = 4


def _build_plan(seq):
    total = seq * (seq + 1) // 2
    starts = [k * seq - (k * (k - 1)) // 2 for k in range(seq)]
    idx_np = np.concatenate(
        [np.arange(n, dtype=np.int32) for n in range(seq, 0, -1)])

    cls_items = {}
    covered = set()
    for k in range(seq):
        s, L = starts[k], seq - k
        h = (8 - s % 8) % 8
        m = ((s + L) // 8) * 8 - s
        j = 0
        while _Q * j + h < m:
            lo, hi = _Q * j + h, min(m, _Q * (j + 1) + h)
            cls_items.setdefault((j, h), []).append((hi - lo, s + lo))
            covered.update(range(s + lo, s + hi, 8))
            j += 1
    boundary = sorted(set(range(0, total, 8)) - covered)

    loads = {c: sum(L for L, _ in v) for c, v in cls_items.items()}
    order = sorted(loads, key=lambda c: -loads[c])
    npairs = len(order) // 2
    pairs = [(order[i], order[len(order) - 1 - i]) for i in range(npairs)]
    wload = [loads[a] + loads[b] for a, b in pairs]
    bassign = [[] for _ in range(npairs)]
    for blk in boundary:
        i = wload.index(min(wload))
        bassign[i].append(blk)
        wload[i] += 16

    idx_rows, bulk_items = [], []
    for P, (c1, c2) in enumerate(pairs):
        row = []
        items = []
        for slot, (j, h) in enumerate((c1, c2)):
            row.extend(range(_Q * j + h, _Q * j + h + _Q))
            items.extend((L, dst, slot * _Q) for L, dst in cls_items[(j, h)])
        for blk in bassign[P]:
            row.extend(int(v) for v in idx_np[blk:blk + 8])
        row.extend([0] * (4 * _Q - len(row)))
        idx_rows.append(row)
        bulk_items.append(items)
    return total, np.asarray(idx_rows, np.int32).reshape(-1), bulk_items, bassign


def kernel(inputs, dist_embedding):
    batch, seq = inputs.shape[0], inputs.shape[1]
    emb = dist_embedding.shape[1]
    assert batch == 2 and seq == 256
    total, idx_flat, bulk_items, bassign = _build_plan(seq)
    nrows = batch * total
    nslots = 4 * _Q

    mesh = plsc.VectorSubcoreMesh(core_axis_name="c", subcore_axis_name="s")

    @functools.partial(
        pl.kernel,
        mesh=mesh,
        out_type=jax.ShapeDtypeStruct((nrows, emb), jnp.float32),
        scratch_types=[
            pltpu.VMEM((nslots,), jnp.int32),
            pltpu.VMEM((2 * _Q, emb), jnp.float32),
            pltpu.VMEM((8 * _BR, emb), jnp.float32),
            pltpu.SemaphoreType.DMA,
            pltpu.SemaphoreType.DMA,
            pltpu.SemaphoreType.DMA,
            pltpu.SemaphoreType.DMA,
        ],
    )
    def _copy_kernel(table_hbm, idx_hbm, out_hbm, idx_v, tbuf, bbuf,
                     sem_stage, sem_bulk, sem_bg, sem_bs):
        wid = lax.axis_index("c") * _NS + lax.axis_index("s")
        pair = wid // batch
        b_off = (wid % batch) * total

        pltpu.sync_copy(idx_hbm.at[pl.ds(pair * nslots, nslots)], idx_v)
        pltpu.async_copy(
            table_hbm.at[idx_v.at[pl.ds(0, 2 * _Q)]], tbuf, sem_stage).wait()

        for P, (items, blocks) in enumerate(zip(bulk_items, bassign)):
            @pl.when(pair == P)
            def _(items=items, blocks=blocks):
                bulk = [
                    pltpu.async_copy(
                        tbuf.at[pl.ds(soff, L)],
                        out_hbm.at[pl.ds(b_off + dst, L)],
                        sem_bulk,
                    )
                    for L, dst, soff in items
                ]
                for r0 in range(0, len(blocks), _BR):
                    blks = blocks[r0:r0 + _BR]
                    n = 8 * len(blks)
                    pltpu.async_copy(
                        table_hbm.at[idx_v.at[pl.ds(2 * _Q + 8 * r0, n)]],
                        bbuf.at[pl.ds(0, n)], sem_bg).wait()
                    scat = [
                        pltpu.async_copy(
                            bbuf.at[pl.ds(8 * q, 8)],
                            out_hbm.at[pl.ds(b_off + dst, 8)],
                            sem_bs,
                        )
                        for q, dst in enumerate(blks)
                    ]
                    for c in scat:
                        c.wait()
                for c in bulk:
                    c.wait()

    out = _copy_kernel(dist_embedding, jnp.asarray(idx_flat))
    return out.reshape(batch, total, emb)
